# speedup vs baseline: 29.6570x; 29.6570x over previous
import functools

import jax
import jax.numpy as jnp
from jax.experimental import pallas as pl

SPARSITY = 0.1
ROWS_PER_BLOCK = 256


def _tc_body(k, x_ref, o_ref):
    xb = x_ref[...]
    n = xb.shape[1]
    scale = jnp.float32(n / k)
    bits = jax.lax.bitcast_convert_type(xb, jnp.int32) & jnp.int32(0x7FFFFFFF)

    def step(_, carry):
        lo, hi = carry
        mid = lo + ((hi - lo) >> 1)
        cnt = jnp.sum((bits >= mid).astype(jnp.int32), axis=1, keepdims=True)
        ge = cnt >= k
        return jnp.where(ge, mid, lo), jnp.where(ge, hi, mid)

    lo0 = jnp.zeros((xb.shape[0], 1), jnp.int32)
    hi0 = jnp.full((xb.shape[0], 1), 0x7F800000, jnp.int32)
    thr, _ = jax.lax.fori_loop(0, 31, step, (lo0, hi0))
    o_ref[...] = jnp.where(bits >= thr, xb * scale, jnp.float32(0.0))


def kernel(x):
    b, s, n = x.shape
    k = max(1, int(n * SPARSITY))
    rows = b * s
    xf = x.reshape(rows, n)
    r = ROWS_PER_BLOCK
    out = pl.pallas_call(
        functools.partial(_tc_body, k),
        grid=(rows // r,),
        in_specs=[pl.BlockSpec((r, n), lambda i: (i, 0))],
        out_specs=pl.BlockSpec((r, n), lambda i: (i, 0)),
        out_shape=jax.ShapeDtypeStruct((rows, n), jnp.float32),
    )(xf)
    return out.reshape(b, s, n)
